# baseline (device time: 416300 ns/iter reference)
import jax
import jax.numpy as jnp
from jax import lax
from jax.experimental import pallas as pl
from jax.experimental.pallas import tpu as pltpu

N_DEV = 8


def _gray(t):
    return jnp.where(t < 4, t, 11 - t)


def kernel(x, w_mat):
    x = x.astype(jnp.bfloat16)
    w_mat = w_mat.astype(jnp.bfloat16)
    m_per, k = x.shape
    _, n_per = w_mat.shape
    m_glob = N_DEV * m_per

    def body(x_ref, w_ref, out_ref, xg_ref, amax_ref,
             send_sems, recv_sems, a_send_sems, a_recv_sems):
        my = lax.axis_index("i")
        r = _gray(my)
        right = _gray(lax.rem(r + 1, N_DEV))
        left = _gray(lax.rem(r + N_DEV - 1, N_DEV))

        barrier_sem = pltpu.get_barrier_semaphore()
        for nbr in (left, right):
            pl.semaphore_signal(
                barrier_sem, inc=1,
                device_id=(nbr,), device_id_type=pl.DeviceIdType.MESH,
            )
        pl.semaphore_wait(barrier_sem, 2)

        def mm(xs):
            z = jax.lax.dot_general(
                xs, w_ref[...],
                dimension_numbers=(((1,), (0,)), ((), ())),
                preferred_element_type=jnp.float32,
            )
            return jnp.maximum(z, 0.0)

        z0 = mm(x_ref[...])
        amax = jnp.max(z0)
        out_ref[pl.ds(my * m_per, m_per), :] = z0

        for h in range(N_DEV - 1):
            src = x_ref if h == 0 else xg_ref.at[h - 1]
            rdma = pltpu.make_async_remote_copy(
                src_ref=src,
                dst_ref=xg_ref.at[h],
                send_sem=send_sems.at[h],
                recv_sem=recv_sems.at[h],
                device_id=(right,),
                device_id_type=pl.DeviceIdType.MESH,
            )
            rdma.start()
            rdma.wait()
            origin = _gray(lax.rem(r - h - 1 + N_DEV, N_DEV))
            z = mm(xg_ref[h])
            amax = jnp.maximum(amax, jnp.max(z))
            out_ref[pl.ds(origin * m_per, m_per), :] = z

        amax_ref[pl.ds(my, 1), :] = jnp.full((1, 128), amax, jnp.float32)
        a_rdmas = []
        for d in range(1, N_DEV):
            tgt = lax.rem(my + d, N_DEV)
            rd = pltpu.make_async_remote_copy(
                src_ref=amax_ref.at[pl.ds(my, 1), :],
                dst_ref=amax_ref.at[pl.ds(my, 1), :],
                send_sem=a_send_sems.at[d - 1],
                recv_sem=a_recv_sems.at[d - 1],
                device_id=(tgt,),
                device_id_type=pl.DeviceIdType.MESH,
            )
            rd.start()
            a_rdmas.append(rd)
        for rd in a_rdmas:
            rd.wait()

        g_amax = jnp.max(amax_ref[...])
        scale = g_amax / 127.0

        for j in range(N_DEV):
            v = out_ref[pl.ds(j * m_per, m_per), :]
            q = jnp.clip(jnp.round(v / scale), 0.0, 127.0)
            out_ref[pl.ds(j * m_per, m_per), :] = q * scale

    return pl.pallas_call(
        body,
        out_shape=jax.ShapeDtypeStruct((m_glob, n_per), jnp.float32),
        in_specs=[
            pl.BlockSpec(memory_space=pltpu.VMEM),
            pl.BlockSpec(memory_space=pltpu.VMEM),
        ],
        out_specs=pl.BlockSpec(memory_space=pltpu.VMEM),
        scratch_shapes=[
            pltpu.VMEM((N_DEV - 1, m_per, k), jnp.bfloat16),
            pltpu.VMEM((N_DEV, 128), jnp.float32),
            pltpu.SemaphoreType.DMA((N_DEV - 1,)),
            pltpu.SemaphoreType.DMA((N_DEV - 1,)),
            pltpu.SemaphoreType.DMA((N_DEV - 1,)),
            pltpu.SemaphoreType.DMA((N_DEV - 1,)),
        ],
        compiler_params=pltpu.CompilerParams(
            collective_id=0,
            vmem_limit_bytes=64 * 1024 * 1024,
        ),
    )(x, w_mat)


# device time: 222672 ns/iter; 1.8696x vs baseline; 1.8696x over previous
import jax
import jax.numpy as jnp
from jax import lax
from jax.experimental import pallas as pl
from jax.experimental.pallas import tpu as pltpu

N_DEV = 8


def _gray(t):
    return jnp.where(t < 4, t, 11 - t)


def kernel(x, w_mat):
    x = x.astype(jnp.bfloat16)
    w_mat = w_mat.astype(jnp.bfloat16)
    m_per, k = x.shape
    _, n_per = w_mat.shape
    m_glob = N_DEV * m_per

    m_half = m_per // 2

    def body(x_ref, w_ref, out_ref, xgt_ref, xgb_ref, amax_ref,
             cw_send_sems, cw_recv_sems, ccw_send_sems, ccw_recv_sems,
             a_send_sems, a_recv_sems):
        my = lax.axis_index("i")
        r = _gray(my)
        right = _gray(lax.rem(r + 1, N_DEV))
        left = _gray(lax.rem(r + N_DEV - 1, N_DEV))

        barrier_sem = pltpu.get_barrier_semaphore()
        for nbr in (left, right):
            pl.semaphore_signal(
                barrier_sem, inc=1,
                device_id=(nbr,), device_id_type=pl.DeviceIdType.MESH,
            )
        pl.semaphore_wait(barrier_sem, 2)

        def mm(xs):
            z = jax.lax.dot_general(
                xs, w_ref[...],
                dimension_numbers=(((1,), (0,)), ((), ())),
                preferred_element_type=jnp.float32,
            )
            return jnp.maximum(z, 0.0)

        def fwd(src, dst, send_sem, recv_sem, dev):
            rd = pltpu.make_async_remote_copy(
                src_ref=src, dst_ref=dst, send_sem=send_sem,
                recv_sem=recv_sem, device_id=(dev,),
                device_id_type=pl.DeviceIdType.MESH,
            )
            rd.start()
            return rd

        cw_rdmas = [fwd(x_ref.at[pl.ds(0, m_half)], xgt_ref.at[0],
                        cw_send_sems.at[0], cw_recv_sems.at[0], right)]
        ccw_rdmas = [fwd(x_ref.at[pl.ds(m_half, m_half)], xgb_ref.at[0],
                         ccw_send_sems.at[0], ccw_recv_sems.at[0], left)]

        z0 = mm(x_ref[...])
        amax = jnp.max(z0)
        out_ref[pl.ds(my * m_per, m_per), :] = z0

        for h in range(N_DEV - 1):
            cw_rdmas[h].wait_recv()
            if h < N_DEV - 2:
                cw_rdmas.append(
                    fwd(xgt_ref.at[h], xgt_ref.at[h + 1],
                        cw_send_sems.at[h + 1], cw_recv_sems.at[h + 1],
                        right))
            ccw_rdmas[h].wait_recv()
            if h < N_DEV - 2:
                ccw_rdmas.append(
                    fwd(xgb_ref.at[h], xgb_ref.at[h + 1],
                        ccw_send_sems.at[h + 1], ccw_recv_sems.at[h + 1],
                        left))
            o_cw = _gray(lax.rem(r - h - 1 + N_DEV, N_DEV))
            o_ccw = _gray(lax.rem(r + h + 1, N_DEV))
            zt = mm(xgt_ref[h])
            amax = jnp.maximum(amax, jnp.max(zt))
            out_ref[pl.ds(o_cw * m_per, m_half), :] = zt
            zb = mm(xgb_ref[h])
            amax = jnp.maximum(amax, jnp.max(zb))
            out_ref[pl.ds(o_ccw * m_per + m_half, m_half), :] = zb

        for rd in cw_rdmas + ccw_rdmas:
            rd.wait_send()

        amax_ref[pl.ds(my, 1), :] = jnp.full((1, 128), amax, jnp.float32)
        a_rdmas = []
        for d in range(1, N_DEV):
            tgt = lax.rem(my + d, N_DEV)
            rd = pltpu.make_async_remote_copy(
                src_ref=amax_ref.at[pl.ds(my, 1), :],
                dst_ref=amax_ref.at[pl.ds(my, 1), :],
                send_sem=a_send_sems.at[d - 1],
                recv_sem=a_recv_sems.at[d - 1],
                device_id=(tgt,),
                device_id_type=pl.DeviceIdType.MESH,
            )
            rd.start()
            a_rdmas.append(rd)
        for rd in a_rdmas:
            rd.wait()

        g_amax = jnp.max(amax_ref[...])
        scale = g_amax / 127.0

        for j in range(N_DEV):
            v = out_ref[pl.ds(j * m_per, m_per), :]
            q = jnp.clip(jnp.round(v / scale), 0.0, 127.0)
            out_ref[pl.ds(j * m_per, m_per), :] = q * scale

    return pl.pallas_call(
        body,
        out_shape=jax.ShapeDtypeStruct((m_glob, n_per), jnp.float32),
        in_specs=[
            pl.BlockSpec(memory_space=pltpu.VMEM),
            pl.BlockSpec(memory_space=pltpu.VMEM),
        ],
        out_specs=pl.BlockSpec(memory_space=pltpu.VMEM),
        scratch_shapes=[
            pltpu.VMEM((N_DEV - 1, m_per // 2, k), jnp.bfloat16),
            pltpu.VMEM((N_DEV - 1, m_per // 2, k), jnp.bfloat16),
            pltpu.VMEM((N_DEV, 128), jnp.float32),
            pltpu.SemaphoreType.DMA((N_DEV - 1,)),
            pltpu.SemaphoreType.DMA((N_DEV - 1,)),
            pltpu.SemaphoreType.DMA((N_DEV - 1,)),
            pltpu.SemaphoreType.DMA((N_DEV - 1,)),
            pltpu.SemaphoreType.DMA((N_DEV - 1,)),
            pltpu.SemaphoreType.DMA((N_DEV - 1,)),
        ],
        compiler_params=pltpu.CompilerParams(
            collective_id=0,
            vmem_limit_bytes=64 * 1024 * 1024,
        ),
    )(x, w_mat)


# device time: 203601 ns/iter; 2.0447x vs baseline; 1.0937x over previous
import jax
import jax.numpy as jnp
from jax import lax
from jax.experimental import pallas as pl
from jax.experimental.pallas import tpu as pltpu

N_DEV = 8


def _gray(t):
    return jnp.where(t < 4, t, 11 - t)


def kernel(x, w_mat):
    x = x.astype(jnp.bfloat16)
    w_mat = w_mat.astype(jnp.bfloat16)
    m_per, k = x.shape
    _, n_per = w_mat.shape
    m_glob = N_DEV * m_per

    N_HOP = 3

    def body(x_ref, w_ref, out_ref, cw_buf, ccw_buf, ch_buf, amax_ref,
             cw_send_sems, cw_recv_sems, ccw_send_sems, ccw_recv_sems,
             ch_send_sem, ch_recv_sem, a_send_sems, a_recv_sems):
        my = lax.axis_index("i")
        r = _gray(my)
        right = _gray(lax.rem(r + 1, N_DEV))
        left = _gray(lax.rem(r + N_DEV - 1, N_DEV))
        even = lax.rem(r, 2) == 0
        partner = _gray(lax.rem(r + jnp.where(even, 3, 5), N_DEV))

        barrier_sem = pltpu.get_barrier_semaphore()
        for nbr in (left, right, partner):
            pl.semaphore_signal(
                barrier_sem, inc=1,
                device_id=(nbr,), device_id_type=pl.DeviceIdType.MESH,
            )
        pl.semaphore_wait(barrier_sem, 3)

        def mm(xs):
            z = jax.lax.dot_general(
                xs, w_ref[...],
                dimension_numbers=(((1,), (0,)), ((), ())),
                preferred_element_type=jnp.float32,
            )
            return jnp.maximum(z, 0.0)

        def fwd(src, dst, send_sem, recv_sem, dev):
            rd = pltpu.make_async_remote_copy(
                src_ref=src, dst_ref=dst, send_sem=send_sem,
                recv_sem=recv_sem, device_id=(dev,),
                device_id_type=pl.DeviceIdType.MESH,
            )
            rd.start()
            return rd

        cw_rdmas = [fwd(x_ref, cw_buf.at[0],
                        cw_send_sems.at[0], cw_recv_sems.at[0], right)]
        ccw_rdmas = [fwd(x_ref, ccw_buf.at[0],
                         ccw_send_sems.at[0], ccw_recv_sems.at[0], left)]

        z0 = mm(x_ref[...])
        amax = jnp.max(z0)
        out_ref[pl.ds(my * m_per, m_per), :] = z0

        for h in range(N_HOP):
            cw_rdmas[h].wait_recv()
            if h < N_HOP - 1:
                cw_rdmas.append(
                    fwd(cw_buf.at[h], cw_buf.at[h + 1],
                        cw_send_sems.at[h + 1], cw_recv_sems.at[h + 1],
                        right))
            if h == 0:
                @pl.when(even)
                def _():
                    fwd(cw_buf.at[0], ch_buf,
                        ch_send_sem, ch_recv_sem, partner)
            ccw_rdmas[h].wait_recv()
            if h < N_HOP - 1:
                ccw_rdmas.append(
                    fwd(ccw_buf.at[h], ccw_buf.at[h + 1],
                        ccw_send_sems.at[h + 1], ccw_recv_sems.at[h + 1],
                        left))
            if h == 0:
                @pl.when(jnp.logical_not(even))
                def _():
                    fwd(ccw_buf.at[0], ch_buf,
                        ch_send_sem, ch_recv_sem, partner)
            o_cw = _gray(lax.rem(r - h - 1 + N_DEV, N_DEV))
            o_ccw = _gray(lax.rem(r + h + 1, N_DEV))
            zt = mm(cw_buf[h])
            amax = jnp.maximum(amax, jnp.max(zt))
            out_ref[pl.ds(o_cw * m_per, m_per), :] = zt
            zb = mm(ccw_buf[h])
            amax = jnp.maximum(amax, jnp.max(zb))
            out_ref[pl.ds(o_ccw * m_per, m_per), :] = zb

        ch_rd = pltpu.make_async_remote_copy(
            src_ref=cw_buf.at[0], dst_ref=ch_buf,
            send_sem=ch_send_sem, recv_sem=ch_recv_sem,
            device_id=(partner,), device_id_type=pl.DeviceIdType.MESH,
        )
        ch_rd.wait_recv()
        o_ch = _gray(lax.rem(r + 4, N_DEV))
        zc = mm(ch_buf[...])
        amax = jnp.maximum(amax, jnp.max(zc))
        out_ref[pl.ds(o_ch * m_per, m_per), :] = zc

        for rd in cw_rdmas + ccw_rdmas + [ch_rd]:
            rd.wait_send()

        amax_ref[pl.ds(my, 1), :] = jnp.full((1, 128), amax, jnp.float32)
        a_rdmas = []
        for d in range(1, N_DEV):
            tgt = lax.rem(my + d, N_DEV)
            rd = pltpu.make_async_remote_copy(
                src_ref=amax_ref.at[pl.ds(my, 1), :],
                dst_ref=amax_ref.at[pl.ds(my, 1), :],
                send_sem=a_send_sems.at[d - 1],
                recv_sem=a_recv_sems.at[d - 1],
                device_id=(tgt,),
                device_id_type=pl.DeviceIdType.MESH,
            )
            rd.start()
            a_rdmas.append(rd)
        for rd in a_rdmas:
            rd.wait()

        g_amax = jnp.max(amax_ref[...])
        scale = g_amax / 127.0
        inv_scale = 127.0 / g_amax

        for j in range(N_DEV):
            v = out_ref[pl.ds(j * m_per, m_per), :]
            q = jnp.minimum(jnp.round(v * inv_scale), 127.0)
            out_ref[pl.ds(j * m_per, m_per), :] = q * scale

    return pl.pallas_call(
        body,
        out_shape=jax.ShapeDtypeStruct((m_glob, n_per), jnp.float32),
        in_specs=[
            pl.BlockSpec(memory_space=pltpu.VMEM),
            pl.BlockSpec(memory_space=pltpu.VMEM),
        ],
        out_specs=pl.BlockSpec(memory_space=pltpu.VMEM),
        scratch_shapes=[
            pltpu.VMEM((N_HOP, m_per, k), jnp.bfloat16),
            pltpu.VMEM((N_HOP, m_per, k), jnp.bfloat16),
            pltpu.VMEM((m_per, k), jnp.bfloat16),
            pltpu.VMEM((N_DEV, 128), jnp.float32),
            pltpu.SemaphoreType.DMA((N_HOP,)),
            pltpu.SemaphoreType.DMA((N_HOP,)),
            pltpu.SemaphoreType.DMA((N_HOP,)),
            pltpu.SemaphoreType.DMA((N_HOP,)),
            pltpu.SemaphoreType.DMA,
            pltpu.SemaphoreType.DMA,
            pltpu.SemaphoreType.DMA((N_DEV - 1,)),
            pltpu.SemaphoreType.DMA((N_DEV - 1,)),
        ],
        compiler_params=pltpu.CompilerParams(
            collective_id=0,
            vmem_limit_bytes=64 * 1024 * 1024,
        ),
    )(x, w_mat)


# device time: 194162 ns/iter; 2.1441x vs baseline; 1.0486x over previous
import jax
import jax.numpy as jnp
from jax import lax
from jax.experimental import pallas as pl
from jax.experimental.pallas import tpu as pltpu

N_DEV = 8
N_HOP = 3
N_Q = 4


def _gray(t):
    return jnp.where(t < 4, t, 11 - t)


def kernel(x, w_mat):
    x = x.astype(jnp.bfloat16)
    w_mat = w_mat.astype(jnp.bfloat16)
    m_per, k = x.shape
    _, n_per = w_mat.shape
    m_glob = N_DEV * m_per
    m_q = m_per // N_Q

    def body(x_ref, w_ref, out_ref, cw_buf, ccw_buf, ch_buf, amax_ref,
             cw_send_sems, cw_recv_sems, ccw_send_sems, ccw_recv_sems,
             ch_send_sems, ch_recv_sems, a_send_sems, a_recv_sems):
        my = lax.axis_index("i")
        r = _gray(my)
        right = _gray(lax.rem(r + 1, N_DEV))
        left = _gray(lax.rem(r + N_DEV - 1, N_DEV))
        even = lax.rem(r, 2) == 0
        partner = _gray(lax.rem(r + jnp.where(even, 3, 5), N_DEV))

        barrier_sem = pltpu.get_barrier_semaphore()
        for nbr in (left, right, partner):
            pl.semaphore_signal(
                barrier_sem, inc=1,
                device_id=(nbr,), device_id_type=pl.DeviceIdType.MESH,
            )
        pl.semaphore_wait(barrier_sem, 3)

        def mm(xs):
            z = jax.lax.dot_general(
                xs, w_ref[...],
                dimension_numbers=(((1,), (0,)), ((), ())),
                preferred_element_type=jnp.float32,
            )
            return jnp.maximum(z, 0.0)

        def fwd(src, dst, send_sem, recv_sem, dev):
            rd = pltpu.make_async_remote_copy(
                src_ref=src, dst_ref=dst, send_sem=send_sem,
                recv_sem=recv_sem, device_id=(dev,),
                device_id_type=pl.DeviceIdType.MESH,
            )
            rd.start()
            return rd

        cw_rdmas = [[fwd(x_ref.at[pl.ds(q * m_q, m_q)], cw_buf.at[0, q],
                         cw_send_sems.at[0, q], cw_recv_sems.at[0, q],
                         right) for q in range(N_Q)]]
        ccw_rdmas = [[fwd(x_ref.at[pl.ds(q * m_q, m_q)], ccw_buf.at[0, q],
                          ccw_send_sems.at[0, q], ccw_recv_sems.at[0, q],
                          left) for q in range(N_Q)]]

        z0 = mm(x_ref[...])
        amax = jnp.max(z0)
        out_ref[pl.ds(my * m_per, m_per), :] = z0

        for h in range(N_HOP):
            if h < N_HOP - 1:
                cw_rdmas.append([])
                ccw_rdmas.append([])
            for q in range(N_Q):
                cw_rdmas[h][q].wait_recv()
                if h < N_HOP - 1:
                    cw_rdmas[h + 1].append(
                        fwd(cw_buf.at[h, q], cw_buf.at[h + 1, q],
                            cw_send_sems.at[h + 1, q],
                            cw_recv_sems.at[h + 1, q], right))
                if h == 0:
                    @pl.when(even)
                    def _():
                        fwd(cw_buf.at[0, q], ch_buf.at[q],
                            ch_send_sems.at[q], ch_recv_sems.at[q],
                            partner)
                ccw_rdmas[h][q].wait_recv()
                if h < N_HOP - 1:
                    ccw_rdmas[h + 1].append(
                        fwd(ccw_buf.at[h, q], ccw_buf.at[h + 1, q],
                            ccw_send_sems.at[h + 1, q],
                            ccw_recv_sems.at[h + 1, q], left))
                if h == 0:
                    @pl.when(jnp.logical_not(even))
                    def _():
                        fwd(ccw_buf.at[0, q], ch_buf.at[q],
                            ch_send_sems.at[q], ch_recv_sems.at[q],
                            partner)
                o_cw = _gray(lax.rem(r - h - 1 + N_DEV, N_DEV))
                o_ccw = _gray(lax.rem(r + h + 1, N_DEV))
                zt = mm(cw_buf[h, q])
                amax = jnp.maximum(amax, jnp.max(zt))
                out_ref[pl.ds(o_cw * m_per + q * m_q, m_q), :] = zt
                zb = mm(ccw_buf[h, q])
                amax = jnp.maximum(amax, jnp.max(zb))
                out_ref[pl.ds(o_ccw * m_per + q * m_q, m_q), :] = zb

        o_ch = _gray(lax.rem(r + 4, N_DEV))
        ch_rdmas = []
        for q in range(N_Q):
            ch_rd = pltpu.make_async_remote_copy(
                src_ref=cw_buf.at[0, q], dst_ref=ch_buf.at[q],
                send_sem=ch_send_sems.at[q], recv_sem=ch_recv_sems.at[q],
                device_id=(partner,), device_id_type=pl.DeviceIdType.MESH,
            )
            ch_rdmas.append(ch_rd)
            ch_rd.wait_recv()
            zc = mm(ch_buf[q])
            amax = jnp.maximum(amax, jnp.max(zc))
            out_ref[pl.ds(o_ch * m_per + q * m_q, m_q), :] = zc

        amax_ref[pl.ds(my, 1), :] = jnp.full((1, 128), amax, jnp.float32)
        a_rdmas = []
        for d in range(1, N_DEV):
            tgt = lax.rem(my + d, N_DEV)
            rd = pltpu.make_async_remote_copy(
                src_ref=amax_ref.at[pl.ds(my, 1), :],
                dst_ref=amax_ref.at[pl.ds(my, 1), :],
                send_sem=a_send_sems.at[d - 1],
                recv_sem=a_recv_sems.at[d - 1],
                device_id=(tgt,),
                device_id_type=pl.DeviceIdType.MESH,
            )
            rd.start()
            a_rdmas.append(rd)
        for rd in a_rdmas:
            rd.wait()

        g_amax = jnp.max(amax_ref[...])
        scale = g_amax / 127.0
        inv_scale = 127.0 / g_amax

        for j in range(N_DEV):
            v = out_ref[pl.ds(j * m_per, m_per), :]
            q = jnp.minimum(jnp.round(v * inv_scale), 127.0)
            out_ref[pl.ds(j * m_per, m_per), :] = q * scale

        for rds in cw_rdmas + ccw_rdmas + [ch_rdmas]:
            for rd in rds:
                rd.wait_send()

    return pl.pallas_call(
        body,
        out_shape=jax.ShapeDtypeStruct((m_glob, n_per), jnp.float32),
        in_specs=[
            pl.BlockSpec(memory_space=pltpu.VMEM),
            pl.BlockSpec(memory_space=pltpu.VMEM),
        ],
        out_specs=pl.BlockSpec(memory_space=pltpu.VMEM),
        scratch_shapes=[
            pltpu.VMEM((N_HOP, N_Q, m_per // N_Q, k), jnp.bfloat16),
            pltpu.VMEM((N_HOP, N_Q, m_per // N_Q, k), jnp.bfloat16),
            pltpu.VMEM((N_Q, m_per // N_Q, k), jnp.bfloat16),
            pltpu.VMEM((N_DEV, 128), jnp.float32),
            pltpu.SemaphoreType.DMA((N_HOP, N_Q)),
            pltpu.SemaphoreType.DMA((N_HOP, N_Q)),
            pltpu.SemaphoreType.DMA((N_HOP, N_Q)),
            pltpu.SemaphoreType.DMA((N_HOP, N_Q)),
            pltpu.SemaphoreType.DMA((N_Q,)),
            pltpu.SemaphoreType.DMA((N_Q,)),
            pltpu.SemaphoreType.DMA((N_DEV - 1,)),
            pltpu.SemaphoreType.DMA((N_DEV - 1,)),
        ],
        compiler_params=pltpu.CompilerParams(
            collective_id=0,
            vmem_limit_bytes=64 * 1024 * 1024,
        ),
    )(x, w_mat)


# device time: 179651 ns/iter; 2.3173x vs baseline; 1.0808x over previous
import jax
import jax.numpy as jnp
from jax import lax
from jax.experimental import pallas as pl
from jax.experimental.pallas import tpu as pltpu

N_DEV = 8
N_HOP = 3
N_Q = 4


def _gray(t):
    return jnp.where(t < 4, t, 11 - t)


def kernel(x, w_mat):
    x = x.astype(jnp.bfloat16)
    m_per, k = x.shape
    _, n_per = w_mat.shape
    m_glob = N_DEV * m_per
    m_q = m_per // N_Q
    KB = 16
    k_b = k // KB

    def body(x_ref, w_ref, out_ref, cw_buf, ccw_buf, ch_buf, w_bf, w_stage,
             out_vmem, amax_ref,
             cw_send_sems, cw_recv_sems, ccw_send_sems, ccw_recv_sems,
             ch_send_sems, ch_recv_sems, a_send_sems, a_recv_sems,
             w_cp_sems, out_cp_sems):
        my = lax.axis_index("i")
        r = _gray(my)
        right = _gray(lax.rem(r + 1, N_DEV))
        left = _gray(lax.rem(r + N_DEV - 1, N_DEV))
        even = lax.rem(r, 2) == 0
        partner = _gray(lax.rem(r + jnp.where(even, 3, 5), N_DEV))

        barrier_sem = pltpu.get_barrier_semaphore()
        for nbr in (left, right, partner):
            pl.semaphore_signal(
                barrier_sem, inc=1,
                device_id=(nbr,), device_id_type=pl.DeviceIdType.MESH,
            )
        pl.semaphore_wait(barrier_sem, 3)

        def mm(xs):
            z = jax.lax.dot_general(
                xs, w_bf[...],
                dimension_numbers=(((1,), (0,)), ((), ())),
                preferred_element_type=jnp.float32,
            )
            return jnp.maximum(z, 0.0)

        def fwd(src, dst, send_sem, recv_sem, dev):
            rd = pltpu.make_async_remote_copy(
                src_ref=src, dst_ref=dst, send_sem=send_sem,
                recv_sem=recv_sem, device_id=(dev,),
                device_id_type=pl.DeviceIdType.MESH,
            )
            rd.start()
            return rd

        cw_rdmas = [[fwd(x_ref.at[pl.ds(q * m_q, m_q)], cw_buf.at[0, q],
                         cw_send_sems.at[0, q], cw_recv_sems.at[0, q],
                         right) for q in range(N_Q)]]
        ccw_rdmas = [[fwd(x_ref.at[pl.ds(q * m_q, m_q)], ccw_buf.at[0, q],
                          ccw_send_sems.at[0, q], ccw_recv_sems.at[0, q],
                          left) for q in range(N_Q)]]

        w_cps = []
        for b in range(KB):
            cp = pltpu.make_async_copy(
                w_ref.at[pl.ds(b * k_b, k_b), :],
                w_stage.at[b % 2], w_cp_sems.at[b % 2])
            cp.start()
            w_cps.append(cp)
            if b > 0:
                w_cps[b - 1].wait()
                w_bf[pl.ds((b - 1) * k_b, k_b), :] = (
                    w_stage[(b - 1) % 2][...].astype(jnp.bfloat16))
        w_cps[KB - 1].wait()
        w_bf[pl.ds((KB - 1) * k_b, k_b), :] = (
            w_stage[(KB - 1) % 2][...].astype(jnp.bfloat16))

        z0 = mm(x_ref[...])
        amax = jnp.max(z0)
        out_vmem[pl.ds(my * m_per, m_per), :] = z0

        for h in range(N_HOP):
            if h < N_HOP - 1:
                cw_rdmas.append([])
                ccw_rdmas.append([])
            for q in range(N_Q):
                cw_rdmas[h][q].wait_recv()
                if h < N_HOP - 1:
                    cw_rdmas[h + 1].append(
                        fwd(cw_buf.at[h, q], cw_buf.at[h + 1, q],
                            cw_send_sems.at[h + 1, q],
                            cw_recv_sems.at[h + 1, q], right))
                if h == 0:
                    @pl.when(even)
                    def _():
                        fwd(cw_buf.at[0, q], ch_buf.at[q],
                            ch_send_sems.at[q], ch_recv_sems.at[q],
                            partner)
                ccw_rdmas[h][q].wait_recv()
                if h < N_HOP - 1:
                    ccw_rdmas[h + 1].append(
                        fwd(ccw_buf.at[h, q], ccw_buf.at[h + 1, q],
                            ccw_send_sems.at[h + 1, q],
                            ccw_recv_sems.at[h + 1, q], left))
                if h == 0:
                    @pl.when(jnp.logical_not(even))
                    def _():
                        fwd(ccw_buf.at[0, q], ch_buf.at[q],
                            ch_send_sems.at[q], ch_recv_sems.at[q],
                            partner)
                o_cw = _gray(lax.rem(r - h - 1 + N_DEV, N_DEV))
                o_ccw = _gray(lax.rem(r + h + 1, N_DEV))
                zt = mm(cw_buf[h, q])
                amax = jnp.maximum(amax, jnp.max(zt))
                out_vmem[pl.ds(o_cw * m_per + q * m_q, m_q), :] = zt
                zb = mm(ccw_buf[h, q])
                amax = jnp.maximum(amax, jnp.max(zb))
                out_vmem[pl.ds(o_ccw * m_per + q * m_q, m_q), :] = zb

        o_ch = _gray(lax.rem(r + 4, N_DEV))
        ch_rdmas = []
        for q in range(N_Q):
            ch_rd = pltpu.make_async_remote_copy(
                src_ref=cw_buf.at[0, q], dst_ref=ch_buf.at[q],
                send_sem=ch_send_sems.at[q], recv_sem=ch_recv_sems.at[q],
                device_id=(partner,), device_id_type=pl.DeviceIdType.MESH,
            )
            ch_rdmas.append(ch_rd)
            ch_rd.wait_recv()
            zc = mm(ch_buf[q])
            amax = jnp.maximum(amax, jnp.max(zc))
            out_vmem[pl.ds(o_ch * m_per + q * m_q, m_q), :] = zc

        amax_ref[pl.ds(my, 1), :] = jnp.full((1, 128), amax, jnp.float32)
        a_rdmas = []
        for d in range(1, N_DEV):
            tgt = lax.rem(my + d, N_DEV)
            rd = pltpu.make_async_remote_copy(
                src_ref=amax_ref.at[pl.ds(my, 1), :],
                dst_ref=amax_ref.at[pl.ds(my, 1), :],
                send_sem=a_send_sems.at[d - 1],
                recv_sem=a_recv_sems.at[d - 1],
                device_id=(tgt,),
                device_id_type=pl.DeviceIdType.MESH,
            )
            rd.start()
            a_rdmas.append(rd)
        for rd in a_rdmas:
            rd.wait()

        g_amax = jnp.max(amax_ref[...])
        scale = g_amax / 127.0
        inv_scale = 127.0 / g_amax

        out_cps = []
        for j in range(N_DEV):
            v = out_vmem[pl.ds(j * m_per, m_per), :]
            q = jnp.minimum(jnp.round(v * inv_scale), 127.0)
            out_vmem[pl.ds(j * m_per, m_per), :] = q * scale
            cp = pltpu.make_async_copy(
                out_vmem.at[pl.ds(j * m_per, m_per), :],
                out_ref.at[pl.ds(j * m_per, m_per), :],
                out_cp_sems.at[j])
            cp.start()
            out_cps.append(cp)
        for cp in out_cps:
            cp.wait()

        for rds in cw_rdmas + ccw_rdmas + [ch_rdmas]:
            for rd in rds:
                rd.wait_send()

    return pl.pallas_call(
        body,
        out_shape=jax.ShapeDtypeStruct((m_glob, n_per), jnp.float32),
        in_specs=[
            pl.BlockSpec(memory_space=pltpu.VMEM),
            pl.BlockSpec(memory_space=pl.ANY),
        ],
        out_specs=pl.BlockSpec(memory_space=pl.ANY),
        scratch_shapes=[
            pltpu.VMEM((N_HOP, N_Q, m_per // N_Q, k), jnp.bfloat16),
            pltpu.VMEM((N_HOP, N_Q, m_per // N_Q, k), jnp.bfloat16),
            pltpu.VMEM((N_Q, m_per // N_Q, k), jnp.bfloat16),
            pltpu.VMEM((k, n_per), jnp.bfloat16),
            pltpu.VMEM((2, k // KB, n_per), jnp.float32),
            pltpu.VMEM((m_glob, n_per), jnp.float32),
            pltpu.VMEM((N_DEV, 128), jnp.float32),
            pltpu.SemaphoreType.DMA((N_HOP, N_Q)),
            pltpu.SemaphoreType.DMA((N_HOP, N_Q)),
            pltpu.SemaphoreType.DMA((N_HOP, N_Q)),
            pltpu.SemaphoreType.DMA((N_HOP, N_Q)),
            pltpu.SemaphoreType.DMA((N_Q,)),
            pltpu.SemaphoreType.DMA((N_Q,)),
            pltpu.SemaphoreType.DMA((N_DEV - 1,)),
            pltpu.SemaphoreType.DMA((N_DEV - 1,)),
            pltpu.SemaphoreType.DMA((2,)),
            pltpu.SemaphoreType.DMA((N_DEV,)),
        ],
        compiler_params=pltpu.CompilerParams(
            collective_id=0,
            vmem_limit_bytes=64 * 1024 * 1024,
        ),
    )(x, w_mat)


# device time: 173169 ns/iter; 2.4040x vs baseline; 1.0374x over previous
import jax
import jax.numpy as jnp
from jax import lax
from jax.experimental import pallas as pl
from jax.experimental.pallas import tpu as pltpu

N_DEV = 8
N_HOP = 3
N_Q = 4


def _gray(t):
    return jnp.where(t < 4, t, 11 - t)


def kernel(x, w_mat):
    x = x.astype(jnp.bfloat16)
    m_per, k = x.shape
    _, n_per = w_mat.shape
    m_glob = N_DEV * m_per
    m_q = m_per // N_Q
    KB = 16
    k_b = k // KB

    def body(x_ref, w_ref, out_ref, cw_buf, ccw_buf, ch_buf, w_bf, w_stage,
             out_vmem, amax_ref,
             cw_send_sems, cw_recv_sems, ccw_send_sems, ccw_recv_sems,
             ch_send_sems, ch_recv_sems, a_send_sems, a_recv_sems,
             w_cp_sems, out_cp_sems):
        my = lax.axis_index("i")
        r = _gray(my)
        right = _gray(lax.rem(r + 1, N_DEV))
        left = _gray(lax.rem(r + N_DEV - 1, N_DEV))
        even = lax.rem(r, 2) == 0
        partner = _gray(lax.rem(r + jnp.where(even, 3, 5), N_DEV))

        barrier_sem = pltpu.get_barrier_semaphore()
        for nbr in (left, right, partner):
            pl.semaphore_signal(
                barrier_sem, inc=1,
                device_id=(nbr,), device_id_type=pl.DeviceIdType.MESH,
            )
        pl.semaphore_wait(barrier_sem, 3)

        def mm(xs):
            z = jax.lax.dot_general(
                xs, w_bf[...],
                dimension_numbers=(((1,), (0,)), ((), ())),
                preferred_element_type=jnp.float32,
            )
            return jnp.maximum(z, 0.0)

        def fwd(src, dst, send_sem, recv_sem, dev):
            rd = pltpu.make_async_remote_copy(
                src_ref=src, dst_ref=dst, send_sem=send_sem,
                recv_sem=recv_sem, device_id=(dev,),
                device_id_type=pl.DeviceIdType.MESH,
            )
            rd.start()
            return rd

        cw_rdmas = [[fwd(x_ref.at[pl.ds(q * m_q, m_q)], cw_buf.at[0, q],
                         cw_send_sems.at[0, q], cw_recv_sems.at[0, q],
                         right) for q in range(N_Q)]]
        ccw_rdmas = [[fwd(x_ref.at[pl.ds(q * m_q, m_q)], ccw_buf.at[0, q],
                          ccw_send_sems.at[0, q], ccw_recv_sems.at[0, q],
                          left) for q in range(N_Q)]]

        w_cps = []
        for b in range(KB):
            cp = pltpu.make_async_copy(
                w_ref.at[pl.ds(b * k_b, k_b), :],
                w_stage.at[b % 2], w_cp_sems.at[b % 2])
            cp.start()
            w_cps.append(cp)
            if b > 0:
                w_cps[b - 1].wait()
                w_bf[pl.ds((b - 1) * k_b, k_b), :] = (
                    w_stage[(b - 1) % 2][...].astype(jnp.bfloat16))
        w_cps[KB - 1].wait()
        w_bf[pl.ds((KB - 1) * k_b, k_b), :] = (
            w_stage[(KB - 1) % 2][...].astype(jnp.bfloat16))

        z0 = mm(x_ref[...])
        amax = jnp.max(z0)
        out_vmem[pl.ds(my * m_per, m_per), :] = z0

        o_ch = _gray(lax.rem(r + 4, N_DEV))
        ch_rdmas = []
        for h in range(N_HOP):
            if h < N_HOP - 1:
                cw_rdmas.append([])
                ccw_rdmas.append([])
            for q in range(N_Q):
                if h == N_HOP - 1:
                    ch_rd = pltpu.make_async_remote_copy(
                        src_ref=cw_buf.at[0, q], dst_ref=ch_buf.at[q],
                        send_sem=ch_send_sems.at[q],
                        recv_sem=ch_recv_sems.at[q],
                        device_id=(partner,),
                        device_id_type=pl.DeviceIdType.MESH,
                    )
                    ch_rdmas.append(ch_rd)
                    ch_rd.wait_recv()
                    zc = mm(ch_buf[q])
                    amax = jnp.maximum(amax, jnp.max(zc))
                    out_vmem[pl.ds(o_ch * m_per + q * m_q, m_q), :] = zc
                cw_rdmas[h][q].wait_recv()
                if h < N_HOP - 1:
                    cw_rdmas[h + 1].append(
                        fwd(cw_buf.at[h, q], cw_buf.at[h + 1, q],
                            cw_send_sems.at[h + 1, q],
                            cw_recv_sems.at[h + 1, q], right))
                if h == 0:
                    @pl.when(even)
                    def _():
                        fwd(cw_buf.at[0, q], ch_buf.at[q],
                            ch_send_sems.at[q], ch_recv_sems.at[q],
                            partner)
                ccw_rdmas[h][q].wait_recv()
                if h < N_HOP - 1:
                    ccw_rdmas[h + 1].append(
                        fwd(ccw_buf.at[h, q], ccw_buf.at[h + 1, q],
                            ccw_send_sems.at[h + 1, q],
                            ccw_recv_sems.at[h + 1, q], left))
                if h == 0:
                    @pl.when(jnp.logical_not(even))
                    def _():
                        fwd(ccw_buf.at[0, q], ch_buf.at[q],
                            ch_send_sems.at[q], ch_recv_sems.at[q],
                            partner)
                o_cw = _gray(lax.rem(r - h - 1 + N_DEV, N_DEV))
                o_ccw = _gray(lax.rem(r + h + 1, N_DEV))
                zt = mm(cw_buf[h, q])
                amax = jnp.maximum(amax, jnp.max(zt))
                out_vmem[pl.ds(o_cw * m_per + q * m_q, m_q), :] = zt
                zb = mm(ccw_buf[h, q])
                amax = jnp.maximum(amax, jnp.max(zb))
                out_vmem[pl.ds(o_ccw * m_per + q * m_q, m_q), :] = zb

        amax_ref[pl.ds(my, 1), :] = jnp.full((1, 128), amax, jnp.float32)
        a_rdmas = []
        for d in range(1, N_DEV):
            tgt = lax.rem(my + d, N_DEV)
            rd = pltpu.make_async_remote_copy(
                src_ref=amax_ref.at[pl.ds(my, 1), :],
                dst_ref=amax_ref.at[pl.ds(my, 1), :],
                send_sem=a_send_sems.at[d - 1],
                recv_sem=a_recv_sems.at[d - 1],
                device_id=(tgt,),
                device_id_type=pl.DeviceIdType.MESH,
            )
            rd.start()
            a_rdmas.append(rd)
        for rd in a_rdmas:
            rd.wait()

        g_amax = jnp.max(amax_ref[...])
        scale = g_amax / 127.0
        inv_scale = 127.0 / g_amax

        out_cps = []
        for j in range(N_DEV):
            v = out_vmem[pl.ds(j * m_per, m_per), :]
            q = jnp.minimum(jnp.round(v * inv_scale), 127.0)
            out_vmem[pl.ds(j * m_per, m_per), :] = q * scale
            cp = pltpu.make_async_copy(
                out_vmem.at[pl.ds(j * m_per, m_per), :],
                out_ref.at[pl.ds(j * m_per, m_per), :],
                out_cp_sems.at[j])
            cp.start()
            out_cps.append(cp)
        for cp in out_cps:
            cp.wait()

        for rds in cw_rdmas + ccw_rdmas + [ch_rdmas]:
            for rd in rds:
                rd.wait_send()

    return pl.pallas_call(
        body,
        out_shape=jax.ShapeDtypeStruct((m_glob, n_per), jnp.float32),
        in_specs=[
            pl.BlockSpec(memory_space=pltpu.VMEM),
            pl.BlockSpec(memory_space=pl.ANY),
        ],
        out_specs=pl.BlockSpec(memory_space=pl.ANY),
        scratch_shapes=[
            pltpu.VMEM((N_HOP, N_Q, m_per // N_Q, k), jnp.bfloat16),
            pltpu.VMEM((N_HOP, N_Q, m_per // N_Q, k), jnp.bfloat16),
            pltpu.VMEM((N_Q, m_per // N_Q, k), jnp.bfloat16),
            pltpu.VMEM((k, n_per), jnp.bfloat16),
            pltpu.VMEM((2, k // KB, n_per), jnp.float32),
            pltpu.VMEM((m_glob, n_per), jnp.float32),
            pltpu.VMEM((N_DEV, 128), jnp.float32),
            pltpu.SemaphoreType.DMA((N_HOP, N_Q)),
            pltpu.SemaphoreType.DMA((N_HOP, N_Q)),
            pltpu.SemaphoreType.DMA((N_HOP, N_Q)),
            pltpu.SemaphoreType.DMA((N_HOP, N_Q)),
            pltpu.SemaphoreType.DMA((N_Q,)),
            pltpu.SemaphoreType.DMA((N_Q,)),
            pltpu.SemaphoreType.DMA((N_DEV - 1,)),
            pltpu.SemaphoreType.DMA((N_DEV - 1,)),
            pltpu.SemaphoreType.DMA((2,)),
            pltpu.SemaphoreType.DMA((N_DEV,)),
        ],
        compiler_params=pltpu.CompilerParams(
            collective_id=0,
            vmem_limit_bytes=64 * 1024 * 1024,
        ),
    )(x, w_mat)
